# baseline (speedup 1.0000x reference)
import functools

import jax
import jax.numpy as jnp
import numpy as np
from jax import lax
from jax.experimental import pallas as pl
from jax.experimental.pallas import tpu as pltpu
from jax.experimental.pallas import tpu_sc as plsc

NUM_EMB = 100000
POS_MAX_LEN = 200
DIM = 128
SCALE = float(np.sqrt(float(DIM)))

NC = 2
NS = 16
NW = NC * NS

B_TOTAL = 1024 * 200
ROWS_PER_W = B_TOTAL // NW
CHUNK = 80
N_CHUNKS = ROWS_PER_W // CHUNK
NBUF = 5
N_OUTER = N_CHUNKS // NBUF
POS_STAGE = 240
PHASES = [(k * CHUNK) % POS_MAX_LEN for k in range(NBUF)]


def _pos_encoding():
    dim_loc = jnp.arange(0, DIM, 2, dtype=jnp.float32)
    pos_loc = jnp.arange(0, POS_MAX_LEN, 1, dtype=jnp.float32)
    denominator = jnp.exp(-(dim_loc / DIM) * jnp.log(jnp.asarray(10000.0)))
    sin_pe = jnp.sin(pos_loc[:, None] * denominator[None, :])
    cos_pe = jnp.cos(pos_loc[:, None] * denominator[None, :])
    pos_enc = jnp.zeros((POS_MAX_LEN, DIM), dtype=jnp.float32)
    pos_enc = pos_enc.at[:, 0::2].set(sin_pe)
    pos_enc = pos_enc.at[:, 1::2].set(cos_pe)
    return pos_enc


@functools.partial(
    pl.kernel,
    mesh=plsc.VectorSubcoreMesh(core_axis_name="c", subcore_axis_name="s"),
    out_type=jax.ShapeDtypeStruct((B_TOTAL, DIM), jnp.float32),
    scratch_types=[
        pltpu.VMEM((N_CHUNKS, CHUNK), jnp.int32),
        pltpu.VMEM((POS_STAGE, DIM), jnp.float32),
        [pltpu.VMEM((CHUNK, DIM), jnp.float32) for _ in range(NBUF)],
        [pltpu.SemaphoreType.DMA for _ in range(NBUF)],
        [pltpu.SemaphoreType.DMA for _ in range(NBUF)],
    ],
)
def _emb_lookup(x_hbm, pos_hbm, table_hbm, out_hbm, idx_v, pos_v, bufs, gsem, osem):
    wid = lax.axis_index("s") * NC + lax.axis_index("c")
    pltpu.sync_copy(x_hbm.at[pl.ds(wid * N_CHUNKS, N_CHUNKS)], idx_v)
    pltpu.sync_copy(pos_hbm, pos_v)
    out_base = wid * ROWS_PER_W

    def gather(k, c):
        return pltpu.make_async_copy(table_hbm.at[idx_v.at[c]], bufs[k], gsem[k])

    def compute(buf, phase):
        def row_body(r, carry):
            for j in range(DIM // 16):
                sl = pl.ds(j * 16, 16)
                buf[r, sl] = buf[r, sl] * SCALE + pos_v[phase + r, sl]
            return carry

        lax.fori_loop(0, CHUNK, row_body, 0)

    for k in range(NBUF):
        gather(k, k).start()

    def out_wait(m):
        pltpu.make_async_copy(
            bufs[m], out_hbm.at[pl.ds(out_base, CHUNK)], osem[m]).wait()

    def outer(i, carry):
        for k in range(NBUF):
            c = i * NBUF + k
            gather(k, c).wait()
            compute(bufs[k], PHASES[k])
            pltpu.async_copy(
                bufs[k], out_hbm.at[pl.ds(out_base + c * CHUNK, CHUNK)],
                osem[k]).start()
            m = (k - 2) % NBUF
            if k >= 2:
                out_wait(m)

                @pl.when(i < N_OUTER - 1)
                def _refill(m=m, i=i, k=k):
                    gather(m, i * NBUF + k + 3).start()
            else:

                @pl.when(i > 0)
                def _drain_refill(m=m, i=i, k=k):
                    out_wait(m)
                    gather(m, i * NBUF + k + 3).start()
        return carry

    lax.fori_loop(0, N_OUTER, outer, 0)
    out_wait((N_CHUNKS - 2) % NBUF)
    out_wait((N_CHUNKS - 1) % NBUF)


def kernel(x, table):
    xf = x.reshape(B_TOTAL // CHUNK, CHUNK).astype(jnp.int32)
    pos = _pos_encoding()
    pos = jnp.concatenate([pos, pos[: POS_STAGE - POS_MAX_LEN]], axis=0)
    out = _emb_lookup(xf, pos, table)
    return out.reshape(1024, POS_MAX_LEN, DIM)


# speedup vs baseline: 1.4573x; 1.4573x over previous
import functools

import jax
import jax.numpy as jnp
import numpy as np
from jax import lax
from jax.experimental import pallas as pl
from jax.experimental.pallas import tpu as pltpu
from jax.experimental.pallas import tpu_sc as plsc

NUM_EMB = 100000
POS_MAX_LEN = 200
DIM = 128
SCALE = float(np.sqrt(float(DIM)))

NC = 2
NS = 16
NW = NC * NS

B_TOTAL = 1024 * 200
ROWS_PER_W = B_TOTAL // NW
CHUNK = 80
N_CHUNKS = ROWS_PER_W // CHUNK
NBUF = 5
N_OUTER = N_CHUNKS // NBUF
POS_STAGE = 240
PHASES = [(k * CHUNK) % POS_MAX_LEN for k in range(NBUF)]


def _pos_encoding():
    dim_loc = jnp.arange(0, DIM, 2, dtype=jnp.float32)
    pos_loc = jnp.arange(0, POS_MAX_LEN, 1, dtype=jnp.float32)
    denominator = jnp.exp(-(dim_loc / DIM) * jnp.log(jnp.asarray(10000.0)))
    sin_pe = jnp.sin(pos_loc[:, None] * denominator[None, :])
    cos_pe = jnp.cos(pos_loc[:, None] * denominator[None, :])
    pos_enc = jnp.zeros((POS_MAX_LEN, DIM), dtype=jnp.float32)
    pos_enc = pos_enc.at[:, 0::2].set(sin_pe)
    pos_enc = pos_enc.at[:, 1::2].set(cos_pe)
    return pos_enc


@functools.partial(
    pl.kernel,
    mesh=plsc.VectorSubcoreMesh(core_axis_name="c", subcore_axis_name="s"),
    out_type=jax.ShapeDtypeStruct((B_TOTAL, DIM), jnp.float32),
    scratch_types=[
        pltpu.VMEM((N_CHUNKS, CHUNK), jnp.int32),
        pltpu.VMEM((POS_STAGE, DIM), jnp.float32),
        [pltpu.VMEM((CHUNK, DIM), jnp.float32) for _ in range(NBUF)],
        [pltpu.SemaphoreType.DMA for _ in range(NBUF)],
        [pltpu.SemaphoreType.DMA for _ in range(NBUF)],
    ],
)
def _emb_lookup(x_hbm, pos_hbm, table_hbm, out_hbm, idx_v, pos_v, bufs, gsem, osem):
    wid = lax.axis_index("s") * NC + lax.axis_index("c")
    idx_cp = pltpu.make_async_copy(
        x_hbm.at[pl.ds(wid * N_CHUNKS, N_CHUNKS)], idx_v, gsem[0])
    pos_cp = pltpu.make_async_copy(pos_hbm, pos_v, osem[0])
    idx_cp.start()
    pos_cp.start()
    idx_cp.wait()
    out_base = wid * ROWS_PER_W

    def gather(k, c):
        return pltpu.make_async_copy(table_hbm.at[idx_v.at[c]], bufs[k], gsem[k])

    def compute(buf, phase):
        def row_body(r, carry):
            for j in range(DIM // 16):
                sl = pl.ds(j * 16, 16)
                buf[r, sl] = buf[r, sl] * SCALE + pos_v[phase + r, sl]
            return carry

        lax.fori_loop(0, CHUNK, row_body, 0)

    for k in range(NBUF):
        gather(k, k).start()
    pos_cp.wait()

    def outer(i, carry):
        out_cps = []
        for k in range(NBUF):
            c = i * NBUF + k
            gather(k, c).wait()
            compute(bufs[k], PHASES[k])
            out_cps.append(pltpu.async_copy(
                bufs[k], out_hbm.at[pl.ds(out_base + c * CHUNK, CHUNK)], osem[k]))
            if k:
                out_cps[k - 1].wait()

                @pl.when(i < N_OUTER - 1)
                def _refill(k=k, i=i):
                    gather(k - 1, (i + 1) * NBUF + k - 1).start()
        out_cps[NBUF - 1].wait()

        @pl.when(i < N_OUTER - 1)
        def _refill_last(i=i):
            gather(NBUF - 1, (i + 1) * NBUF + NBUF - 1).start()

        return carry

    lax.fori_loop(0, N_OUTER, outer, 0)


def kernel(x, table):
    xf = x.reshape(B_TOTAL // CHUNK, CHUNK).astype(jnp.int32)
    pos = _pos_encoding()
    pos = jnp.concatenate([pos, pos[: POS_STAGE - POS_MAX_LEN]], axis=0)
    out = _emb_lookup(xf, pos, table)
    return out.reshape(1024, POS_MAX_LEN, DIM)
